# baseline (speedup 1.0000x reference)
import functools

import jax
import jax.numpy as jnp
from jax import lax
from jax.experimental import pallas as pl
from jax.experimental.pallas import tpu as pltpu
from jax.experimental.pallas import tpu_sc as plsc

B = 16384
D = 128
NC = 2
NS = 16
NW = NC * NS
BPW = B // NW
C = 128
NCH = BPW // C
LJ = D // 16
_BITREV = [int(f"{k:04b}"[::-1], 2) for k in range(16)]


def _permute(x, idx):
    dnums = lax.GatherDimensionNumbers(
        offset_dims=(), collapsed_slice_dims=(0,), start_index_map=(0,))
    return lax.gather(x, idx[:, None], dnums, slice_sizes=(1,),
                      mode=lax.GatherScatterMode.PROMISE_IN_BOUNDS)


NBUF = 3


def _sc_body(ent_hbm, idx_h_hbm, idx_t_hbm, rel_hbm, out_hbm,
             idx_h_v, idx_t_v, h0, h1, h2, t0, t1, t2, rel_v, out_v,
             sh0, sh1, sh2, st0, st1, st2):
    wid = lax.axis_index("c") * NS + lax.axis_index("s")
    base = wid * BPW

    pltpu.sync_copy(idx_h_hbm.at[wid], idx_h_v)
    pltpu.sync_copy(idx_t_hbm.at[wid], idx_t_v)
    pltpu.sync_copy(rel_hbm, rel_v)
    rel_regs = [rel_v[pl.ds(16 * j, 16)] for j in range(LJ)]

    lane = lax.iota(jnp.int32, 16)
    folds = [((lane & ~(gw - 1)) | ((lane + gw // 2) & (gw - 1)), lane & (gw // 2) == 0)
             for gw in (16, 8, 4, 2)]

    hbuf, tbuf = [h0, h1, h2], [t0, t1, t2]
    shs, sts = [sh0, sh1, sh2], [st0, st1, st2]
    cps = {}

    def issue(c):
        p = c % NBUF
        cp_h = pltpu.make_async_copy(ent_hbm.at[idx_h_v.at[c]], hbuf[p], shs[p])
        cp_t = pltpu.make_async_copy(ent_hbm.at[idx_t_v.at[c]], tbuf[p], sts[p])
        cp_h.start()
        cp_t.start()
        cps[c] = (cp_h, cp_t)

    for c in range(min(NBUF - 1, NCH)):
        issue(c)
    for chunk in range(NCH):
        if chunk + NBUF - 1 < NCH:
            issue(chunk + NBUF - 1)
        cp_h, cp_t = cps.pop(chunk)
        cp_h.wait()
        cp_t.wait()
        p = chunk % NBUF
        hv_ref, tv_ref = hbuf[p], tbuf[p]

        @plsc.parallel_loop(0, C // 16)
        def group_body(g, hv_ref=hv_ref, tv_ref=tv_ref, chunk=chunk):
            row0 = g * 16
            vecs = []
            for k in _BITREV:
                acc = jnp.zeros((16,), jnp.float32)
                for j in range(LJ):
                    hv = hv_ref[row0 + k, pl.ds(16 * j, 16)]
                    tv = tv_ref[row0 + k, pl.ds(16 * j, 16)]
                    acc = acc + hv * tv * rel_regs[j]
                vecs.append(acc)
            for fidx, mask in folds:
                nxt = []
                for i in range(0, len(vecs), 2):
                    xf = vecs[i] + _permute(vecs[i], fidx)
                    yf = vecs[i + 1] + _permute(vecs[i + 1], fidx)
                    nxt.append(jnp.where(mask, xf, yf))
                vecs = nxt
            out_v[pl.ds(chunk * C + row0, 16)] = vecs[0]

    pltpu.sync_copy(out_v, out_hbm.at[pl.ds(base, BPW)])


@jax.jit
def _distmult_sc(ent_embeddings, idx_h, idx_t, rel_row):
    mesh = plsc.VectorSubcoreMesh(core_axis_name="c", subcore_axis_name="s")
    fn = pl.kernel(
        _sc_body,
        out_type=jax.ShapeDtypeStruct((B,), jnp.float32),
        mesh=mesh,
        scratch_types=[
            pltpu.VMEM((NCH, C), jnp.int32),
            pltpu.VMEM((NCH, C), jnp.int32),
            pltpu.VMEM((C, D), jnp.float32),
            pltpu.VMEM((C, D), jnp.float32),
            pltpu.VMEM((C, D), jnp.float32),
            pltpu.VMEM((C, D), jnp.float32),
            pltpu.VMEM((C, D), jnp.float32),
            pltpu.VMEM((C, D), jnp.float32),
            pltpu.VMEM((D,), jnp.float32),
            pltpu.VMEM((BPW,), jnp.float32),
            pltpu.SemaphoreType.DMA,
            pltpu.SemaphoreType.DMA,
            pltpu.SemaphoreType.DMA,
            pltpu.SemaphoreType.DMA,
            pltpu.SemaphoreType.DMA,
            pltpu.SemaphoreType.DMA,
        ],
    )
    return fn(ent_embeddings, idx_h, idx_t, rel_row)


def kernel(predict_h, predict_t, r, ent_embeddings, rel_embeddings):
    rel_row = lax.dynamic_index_in_dim(rel_embeddings, r, axis=0, keepdims=False)
    idx_h = predict_h.reshape(NW, NCH, C)
    idx_t = predict_t.reshape(NW, NCH, C)
    return _distmult_sc(ent_embeddings, idx_h, idx_t, rel_row)


# speedup vs baseline: 1.0019x; 1.0019x over previous
import functools

import jax
import jax.numpy as jnp
from jax import lax
from jax.experimental import pallas as pl
from jax.experimental.pallas import tpu as pltpu
from jax.experimental.pallas import tpu_sc as plsc

B = 16384
D = 128
NC = 2
NS = 16
NW = NC * NS
BPW = B // NW
C = 128
NCH = BPW // C
LJ = D // 16
_BITREV = [int(f"{k:04b}"[::-1], 2) for k in range(16)]


def _permute(x, idx):
    dnums = lax.GatherDimensionNumbers(
        offset_dims=(), collapsed_slice_dims=(0,), start_index_map=(0,))
    return lax.gather(x, idx[:, None], dnums, slice_sizes=(1,),
                      mode=lax.GatherScatterMode.PROMISE_IN_BOUNDS)


NBUF = 3


def _sc_body(ent_hbm, idx_h_hbm, idx_t_hbm, rel_hbm, out_hbm,
             idx_h_v, idx_t_v, h0, h1, h2, t0, t1, t2, rel_v, out_v,
             sh0, sh1, sh2, st0, st1, st2):
    wid = lax.axis_index("c") * NS + lax.axis_index("s")
    base = wid * BPW

    pltpu.sync_copy(idx_h_hbm.at[wid], idx_h_v)
    pltpu.sync_copy(idx_t_hbm.at[wid], idx_t_v)
    pltpu.sync_copy(rel_hbm, rel_v)
    rel_regs = [rel_v[pl.ds(16 * j, 16)] for j in range(LJ)]

    lane = lax.iota(jnp.int32, 16)
    folds = [((lane & ~(gw - 1)) | ((lane + gw // 2) & (gw - 1)), lane & (gw // 2) == 0)
             for gw in (16, 8, 4, 2)]

    hbuf, tbuf = [h0, h1, h2], [t0, t1, t2]
    shs, sts = [sh0, sh1, sh2], [st0, st1, st2]
    cps = {}

    def issue(c):
        p = c % NBUF
        cp_h = pltpu.make_async_copy(ent_hbm.at[idx_h_v.at[c]], hbuf[p], shs[p])
        cp_t = pltpu.make_async_copy(ent_hbm.at[idx_t_v.at[c]], tbuf[p], sts[p])
        cp_h.start()
        cp_t.start()
        cps[c] = (cp_h, cp_t)

    for c in range(min(NBUF - 1, NCH)):
        issue(c)
    for chunk in range(NCH):
        if chunk + NBUF - 1 < NCH:
            issue(chunk + NBUF - 1)
        cp_h, cp_t = cps.pop(chunk)
        cp_h.wait()
        cp_t.wait()
        p = chunk % NBUF
        hv_ref, tv_ref = hbuf[p], tbuf[p]

        @plsc.parallel_loop(0, C // 16)
        def group_body(g, hv_ref=hv_ref, tv_ref=tv_ref, chunk=chunk):
            row0 = g * 16
            vecs = [jnp.zeros((16,), jnp.float32) for _ in range(16)]
            for j in range(LJ):
                rj = rel_regs[j]
                for k in range(16):
                    hv = hv_ref[row0 + _BITREV[k], pl.ds(16 * j, 16)]
                    tv = tv_ref[row0 + _BITREV[k], pl.ds(16 * j, 16)]
                    vecs[k] = vecs[k] + hv * tv * rj
            for fidx, mask in folds:
                nxt = []
                for i in range(0, len(vecs), 2):
                    xf = vecs[i] + _permute(vecs[i], fidx)
                    yf = vecs[i + 1] + _permute(vecs[i + 1], fidx)
                    nxt.append(jnp.where(mask, xf, yf))
                vecs = nxt
            out_v[pl.ds(chunk * C + row0, 16)] = vecs[0]

    pltpu.sync_copy(out_v, out_hbm.at[pl.ds(base, BPW)])


@jax.jit
def _distmult_sc(ent_embeddings, idx_h, idx_t, rel_row):
    mesh = plsc.VectorSubcoreMesh(core_axis_name="c", subcore_axis_name="s")
    fn = pl.kernel(
        _sc_body,
        out_type=jax.ShapeDtypeStruct((B,), jnp.float32),
        mesh=mesh,
        scratch_types=[
            pltpu.VMEM((NCH, C), jnp.int32),
            pltpu.VMEM((NCH, C), jnp.int32),
            pltpu.VMEM((C, D), jnp.float32),
            pltpu.VMEM((C, D), jnp.float32),
            pltpu.VMEM((C, D), jnp.float32),
            pltpu.VMEM((C, D), jnp.float32),
            pltpu.VMEM((C, D), jnp.float32),
            pltpu.VMEM((C, D), jnp.float32),
            pltpu.VMEM((D,), jnp.float32),
            pltpu.VMEM((BPW,), jnp.float32),
            pltpu.SemaphoreType.DMA,
            pltpu.SemaphoreType.DMA,
            pltpu.SemaphoreType.DMA,
            pltpu.SemaphoreType.DMA,
            pltpu.SemaphoreType.DMA,
            pltpu.SemaphoreType.DMA,
        ],
    )
    return fn(ent_embeddings, idx_h, idx_t, rel_row)


def kernel(predict_h, predict_t, r, ent_embeddings, rel_embeddings):
    rel_row = lax.dynamic_index_in_dim(rel_embeddings, r, axis=0, keepdims=False)
    idx_h = predict_h.reshape(NW, NCH, C)
    idx_t = predict_t.reshape(NW, NCH, C)
    return _distmult_sc(ent_embeddings, idx_h, idx_t, rel_row)


# speedup vs baseline: 1.2043x; 1.2021x over previous
import functools

import jax
import jax.numpy as jnp
from jax import lax
from jax.experimental import pallas as pl
from jax.experimental.pallas import tpu as pltpu
from jax.experimental.pallas import tpu_sc as plsc

B = 16384
D = 128
NC = 2
NS = 16
NW = NC * NS
BPW = B // NW
C = 128
NCH = BPW // C
LJ = D // 16
_BITREV = [int(f"{k:04b}"[::-1], 2) for k in range(16)]


def _permute(x, idx):
    dnums = lax.GatherDimensionNumbers(
        offset_dims=(), collapsed_slice_dims=(0,), start_index_map=(0,))
    return lax.gather(x, idx[:, None], dnums, slice_sizes=(1,),
                      mode=lax.GatherScatterMode.PROMISE_IN_BOUNDS)


NBUF = 3


def _sc_body(ent_hbm, idx_h_hbm, idx_t_hbm, rel_hbm, out_hbm,
             idx_h_v, idx_t_v, h0, h1, h2, t0, t1, t2, rel_v, out_v,
             sh0, sh1, sh2, st0, st1, st2):
    wid = lax.axis_index("c") * NS + lax.axis_index("s")
    base = wid * BPW

    pltpu.sync_copy(idx_h_hbm.at[wid], idx_h_v)
    pltpu.sync_copy(idx_t_hbm.at[wid], idx_t_v)
    pltpu.sync_copy(rel_hbm, rel_v)
    rel_regs = [rel_v[pl.ds(16 * j, 16)] for j in range(LJ)]

    lane = lax.iota(jnp.int32, 16)
    folds = [((lane & ~(gw - 1)) | ((lane + gw // 2) & (gw - 1)), lane & (gw // 2) == 0)
             for gw in (16, 8, 4, 2)]

    hbuf, tbuf = [h0, h1, h2], [t0, t1, t2]
    shs, sts = [sh0, sh1, sh2], [st0, st1, st2]
    cps = {}

    def issue(c):
        p = c % NBUF
        cp_h = pltpu.make_async_copy(ent_hbm.at[idx_h_v.at[c]], hbuf[p], shs[p])
        cp_t = pltpu.make_async_copy(ent_hbm.at[idx_t_v.at[c]], tbuf[p], sts[p])
        cp_h.start()
        cp_t.start()
        cps[c] = (cp_h, cp_t)

    for c in range(min(NBUF - 1, NCH)):
        issue(c)
    for chunk in range(NCH):
        if chunk + NBUF - 1 < NCH:
            issue(chunk + NBUF - 1)
        cp_h, cp_t = cps.pop(chunk)
        cp_h.wait()
        cp_t.wait()
        p = chunk % NBUF
        hv_ref, tv_ref = hbuf[p], tbuf[p]

        @plsc.parallel_loop(0, C // 16)
        def group_body(g, hv_ref=hv_ref, tv_ref=tv_ref, chunk=chunk):
            row0 = g * 16
            def jbody(j, vecs):
                col = pl.ds(j * 16, 16)
                rj = rel_v[col]
                return [vecs[k]
                        + hv_ref[row0 + _BITREV[k], col]
                        * tv_ref[row0 + _BITREV[k], col] * rj
                        for k in range(16)]

            vecs = lax.fori_loop(
                0, LJ, jbody, [jnp.zeros((16,), jnp.float32)] * 16)
            for fidx, mask in folds:
                nxt = []
                for i in range(0, len(vecs), 2):
                    xf = vecs[i] + _permute(vecs[i], fidx)
                    yf = vecs[i + 1] + _permute(vecs[i + 1], fidx)
                    nxt.append(jnp.where(mask, xf, yf))
                vecs = nxt
            out_v[pl.ds(chunk * C + row0, 16)] = vecs[0]

    pltpu.sync_copy(out_v, out_hbm.at[pl.ds(base, BPW)])


@jax.jit
def _distmult_sc(ent_embeddings, idx_h, idx_t, rel_row):
    mesh = plsc.VectorSubcoreMesh(core_axis_name="c", subcore_axis_name="s")
    fn = pl.kernel(
        _sc_body,
        out_type=jax.ShapeDtypeStruct((B,), jnp.float32),
        mesh=mesh,
        scratch_types=[
            pltpu.VMEM((NCH, C), jnp.int32),
            pltpu.VMEM((NCH, C), jnp.int32),
            pltpu.VMEM((C, D), jnp.float32),
            pltpu.VMEM((C, D), jnp.float32),
            pltpu.VMEM((C, D), jnp.float32),
            pltpu.VMEM((C, D), jnp.float32),
            pltpu.VMEM((C, D), jnp.float32),
            pltpu.VMEM((C, D), jnp.float32),
            pltpu.VMEM((D,), jnp.float32),
            pltpu.VMEM((BPW,), jnp.float32),
            pltpu.SemaphoreType.DMA,
            pltpu.SemaphoreType.DMA,
            pltpu.SemaphoreType.DMA,
            pltpu.SemaphoreType.DMA,
            pltpu.SemaphoreType.DMA,
            pltpu.SemaphoreType.DMA,
        ],
    )
    return fn(ent_embeddings, idx_h, idx_t, rel_row)


def kernel(predict_h, predict_t, r, ent_embeddings, rel_embeddings):
    rel_row = lax.dynamic_index_in_dim(rel_embeddings, r, axis=0, keepdims=False)
    idx_h = predict_h.reshape(NW, NCH, C)
    idx_t = predict_t.reshape(NW, NCH, C)
    return _distmult_sc(ent_embeddings, idx_h, idx_t, rel_row)


# speedup vs baseline: 1.2146x; 1.0085x over previous
import functools

import jax
import jax.numpy as jnp
from jax import lax
from jax.experimental import pallas as pl
from jax.experimental.pallas import tpu as pltpu
from jax.experimental.pallas import tpu_sc as plsc

B = 16384
D = 128
NC = 2
NS = 16
NW = NC * NS
BPW = B // NW
C = 128
NCH = BPW // C
LJ = D // 16
_BITREV = [int(f"{k:04b}"[::-1], 2) for k in range(16)]


def _permute(x, idx):
    dnums = lax.GatherDimensionNumbers(
        offset_dims=(), collapsed_slice_dims=(0,), start_index_map=(0,))
    return lax.gather(x, idx[:, None], dnums, slice_sizes=(1,),
                      mode=lax.GatherScatterMode.PROMISE_IN_BOUNDS)


def _sc_body(ent_hbm, idx_h_hbm, idx_t_hbm, rel_hbm, r_hbm, out_hbm,
             idx_h_v, idx_t_v, h0, h1, t0, t1, rel_v, r_v, out_v,
             sh0, sh1, st0, st1):
    wid = lax.axis_index("c") * NS + lax.axis_index("s")
    base = wid * BPW

    pltpu.sync_copy(idx_h_hbm.at[wid], idx_h_v)
    pltpu.sync_copy(idx_t_hbm.at[wid], idx_t_v)
    pltpu.sync_copy(r_hbm, r_v)
    rv = r_v[pl.ds(0, 16)][0]
    pltpu.sync_copy(rel_hbm.at[pl.ds(rv, 1)], rel_v)
    rel_regs = [rel_v[0, pl.ds(16 * j, 16)] for j in range(LJ)]

    lane = lax.iota(jnp.int32, 16)
    folds = [((lane & ~(gw - 1)) | ((lane + gw // 2) & (gw - 1)), lane & (gw // 2) == 0)
             for gw in (16, 8, 4, 2)]

    hbuf, tbuf = [h0, h1], [t0, t1]
    shs, sts = [sh0, sh1], [st0, st1]

    def issue(c, par):
        pltpu.make_async_copy(ent_hbm.at[idx_h_v.at[c]], hbuf[par], shs[par]).start()
        pltpu.make_async_copy(ent_hbm.at[idx_t_v.at[c]], tbuf[par], sts[par]).start()

    issue(0, 0)
    issue(1, 1)

    def compute(c, hv_ref, tv_ref):
        @plsc.parallel_loop(0, C // 16)
        def group_body(g):
            row0 = g * 16

            def jbody(j, vecs):
                col = pl.ds(j * 16, 16)
                rj = rel_v[0, col]
                return [vecs[k]
                        + hv_ref[row0 + _BITREV[k], col]
                        * tv_ref[row0 + _BITREV[k], col] * rj
                        for k in range(16)]

            vecs = lax.fori_loop(
                0, LJ, jbody, [jnp.zeros((16,), jnp.float32)] * 16)
            for fidx, mask in folds:
                nxt = []
                for i in range(0, len(vecs), 2):
                    xf = vecs[i] + _permute(vecs[i], fidx)
                    yf = vecs[i + 1] + _permute(vecs[i + 1], fidx)
                    nxt.append(jnp.where(mask, xf, yf))
                vecs = nxt
            out_v[pl.ds(c * C + row0, 16)] = vecs[0]

    def pair_body(cc, carry):
        for par in range(2):
            c = cc * 2 + par
            pltpu.make_async_copy(ent_hbm.at[idx_h_v.at[c]], hbuf[par], shs[par]).wait()
            pltpu.make_async_copy(ent_hbm.at[idx_t_v.at[c]], tbuf[par], sts[par]).wait()
            compute(c, hbuf[par], tbuf[par])

            @pl.when(c + 2 < NCH)
            def _(c=c, par=par):
                issue(c + 2, par)
        return carry

    lax.fori_loop(0, NCH // 2, pair_body, 0)

    pltpu.sync_copy(out_v, out_hbm.at[pl.ds(base, BPW)])


@jax.jit
def _distmult_sc(ent_embeddings, idx_h, idx_t, rel_embeddings, r_arr):
    mesh = plsc.VectorSubcoreMesh(core_axis_name="c", subcore_axis_name="s")
    fn = pl.kernel(
        _sc_body,
        out_type=jax.ShapeDtypeStruct((B,), jnp.float32),
        mesh=mesh,
        scratch_types=[
            pltpu.VMEM((NCH, C), jnp.int32),
            pltpu.VMEM((NCH, C), jnp.int32),
            pltpu.VMEM((C, D), jnp.float32),
            pltpu.VMEM((C, D), jnp.float32),
            pltpu.VMEM((C, D), jnp.float32),
            pltpu.VMEM((C, D), jnp.float32),
            pltpu.VMEM((1, D), jnp.float32),
            pltpu.VMEM((16,), jnp.int32),
            pltpu.VMEM((BPW,), jnp.float32),
            pltpu.SemaphoreType.DMA,
            pltpu.SemaphoreType.DMA,
            pltpu.SemaphoreType.DMA,
            pltpu.SemaphoreType.DMA,
        ],
    )
    return fn(ent_embeddings, idx_h, idx_t, rel_embeddings, r_arr)


def kernel(predict_h, predict_t, r, ent_embeddings, rel_embeddings):
    r_arr = jnp.full((16,), r, dtype=jnp.int32)
    idx_h = predict_h.reshape(NW, NCH, C)
    idx_t = predict_t.reshape(NW, NCH, C)
    return _distmult_sc(ent_embeddings, idx_h, idx_t, rel_embeddings, r_arr)


# speedup vs baseline: 1.2387x; 1.0199x over previous
import functools

import jax
import jax.numpy as jnp
from jax import lax
from jax.experimental import pallas as pl
from jax.experimental.pallas import tpu as pltpu
from jax.experimental.pallas import tpu_sc as plsc

B = 16384
D = 128
NC = 2
NS = 16
NW = NC * NS
BPW = B // NW
C = 64
NCH = BPW // C
NBUF = 4
LJ = D // 16
_BITREV = [int(f"{k:04b}"[::-1], 2) for k in range(16)]


def _permute(x, idx):
    dnums = lax.GatherDimensionNumbers(
        offset_dims=(), collapsed_slice_dims=(0,), start_index_map=(0,))
    return lax.gather(x, idx[:, None], dnums, slice_sizes=(1,),
                      mode=lax.GatherScatterMode.PROMISE_IN_BOUNDS)


def _sc_body(ent_hbm, idx_h_hbm, idx_t_hbm, rel_hbm, r_hbm, out_hbm,
             idx_h_v, idx_t_v, h0, h1, h2, h3, t0, t1, t2, t3,
             rel_v, r_v, out_v,
             sh0, sh1, sh2, sh3, st0, st1, st2, st3):
    wid = lax.axis_index("c") * NS + lax.axis_index("s")
    base = wid * BPW

    cp_ih = pltpu.make_async_copy(idx_h_hbm.at[wid], idx_h_v, sh0)
    cp_it = pltpu.make_async_copy(idx_t_hbm.at[wid], idx_t_v, st0)
    cp_r = pltpu.make_async_copy(r_hbm, r_v, sh1)
    cp_ih.start()
    cp_it.start()
    cp_r.start()
    cp_ih.wait()
    cp_it.wait()

    hbuf, tbuf = [h0, h1, h2, h3], [t0, t1, t2, t3]
    shs, sts = [sh0, sh1, sh2, sh3], [st0, st1, st2, st3]

    def issue(c, par):
        pltpu.make_async_copy(ent_hbm.at[idx_h_v.at[c]], hbuf[par], shs[par]).start()
        pltpu.make_async_copy(ent_hbm.at[idx_t_v.at[c]], tbuf[par], sts[par]).start()

    cp_r.wait()
    for par in range(NBUF):
        issue(par, par)
    rv = r_v[pl.ds(0, 16)][0]
    pltpu.sync_copy(rel_hbm.at[pl.ds(rv, 1)], rel_v)

    lane = lax.iota(jnp.int32, 16)
    folds = [((lane & ~(gw - 1)) | ((lane + gw // 2) & (gw - 1)), lane & (gw // 2) == 0)
             for gw in (16, 8, 4, 2)]

    def compute(c, hv_ref, tv_ref):
        @plsc.parallel_loop(0, C // 16)
        def group_body(g):
            row0 = g * 16

            def jbody(j, vecs):
                col = pl.ds(j * 16, 16)
                rj = rel_v[0, col]
                return [vecs[k]
                        + hv_ref[row0 + _BITREV[k], col]
                        * tv_ref[row0 + _BITREV[k], col] * rj
                        for k in range(16)]

            vecs = lax.fori_loop(
                0, LJ, jbody, [jnp.zeros((16,), jnp.float32)] * 16)
            for fidx, mask in folds:
                nxt = []
                for i in range(0, len(vecs), 2):
                    xf = vecs[i] + _permute(vecs[i], fidx)
                    yf = vecs[i + 1] + _permute(vecs[i + 1], fidx)
                    nxt.append(jnp.where(mask, xf, yf))
                vecs = nxt
            out_v[pl.ds(c * C + row0, 16)] = vecs[0]

    def ring_body(qq, carry):
        for par in range(NBUF):
            c = qq * NBUF + par
            pltpu.make_async_copy(ent_hbm.at[idx_h_v.at[c]], hbuf[par], shs[par]).wait()
            pltpu.make_async_copy(ent_hbm.at[idx_t_v.at[c]], tbuf[par], sts[par]).wait()
            compute(c, hbuf[par], tbuf[par])

            @pl.when(c + NBUF < NCH)
            def _(c=c, par=par):
                issue(c + NBUF, par)
        return carry

    lax.fori_loop(0, NCH // NBUF, ring_body, 0)

    pltpu.sync_copy(out_v, out_hbm.at[pl.ds(base, BPW)])


@jax.jit
def _distmult_sc(ent_embeddings, idx_h, idx_t, rel_embeddings, r_arr):
    mesh = plsc.VectorSubcoreMesh(core_axis_name="c", subcore_axis_name="s")
    fn = pl.kernel(
        _sc_body,
        out_type=jax.ShapeDtypeStruct((B,), jnp.float32),
        mesh=mesh,
        scratch_types=(
            [pltpu.VMEM((NCH, C), jnp.int32)] * 2
            + [pltpu.VMEM((C, D), jnp.float32)] * (2 * NBUF)
            + [pltpu.VMEM((1, D), jnp.float32),
               pltpu.VMEM((16,), jnp.int32),
               pltpu.VMEM((BPW,), jnp.float32)]
            + [pltpu.SemaphoreType.DMA] * (2 * NBUF)
        ),
    )
    return fn(ent_embeddings, idx_h, idx_t, rel_embeddings, r_arr)


def kernel(predict_h, predict_t, r, ent_embeddings, rel_embeddings):
    r_arr = jnp.full((16,), r, dtype=jnp.int32)
    idx_h = predict_h.reshape(NW, NCH, C)
    idx_t = predict_t.reshape(NW, NCH, C)
    return _distmult_sc(ent_embeddings, idx_h, idx_t, rel_embeddings, r_arr)


# speedup vs baseline: 1.2897x; 1.0411x over previous
import functools

import jax
import jax.numpy as jnp
from jax import lax
from jax.experimental import pallas as pl
from jax.experimental.pallas import tpu as pltpu
from jax.experimental.pallas import tpu_sc as plsc

B = 16384
D = 128
NC = 2
NS = 16
NW = NC * NS
BPW = B // NW
C = 64
NCH = BPW // C
NBUF = 4
LJ = D // 16
_BITREV = [int(f"{k:04b}"[::-1], 2) for k in range(16)]


def _permute(x, idx):
    dnums = lax.GatherDimensionNumbers(
        offset_dims=(), collapsed_slice_dims=(0,), start_index_map=(0,))
    return lax.gather(x, idx[:, None], dnums, slice_sizes=(1,),
                      mode=lax.GatherScatterMode.PROMISE_IN_BOUNDS)


def _sc_body(ent_hbm, idx_h_hbm, idx_t_hbm, rel_hbm, r_hbm, out_hbm,
             idx_h_v, idx_t_v, h0, h1, h2, h3, t0, t1, t2, t3,
             rel_v, r_v, out_v,
             sh0, sh1, sh2, sh3, st0, st1, st2, st3):
    wid = lax.axis_index("c") * NS + lax.axis_index("s")
    base = wid * BPW

    cp_ih = pltpu.make_async_copy(idx_h_hbm.at[pl.ds(base, BPW)], idx_h_v, sh0)
    cp_it = pltpu.make_async_copy(idx_t_hbm.at[pl.ds(base, BPW)], idx_t_v, st0)
    cp_r = pltpu.make_async_copy(r_hbm, r_v, sh1)
    cp_ih.start()
    cp_it.start()
    cp_r.start()
    cp_ih.wait()
    cp_it.wait()

    hbuf, tbuf = [h0, h1, h2, h3], [t0, t1, t2, t3]
    shs, sts = [sh0, sh1, sh2, sh3], [st0, st1, st2, st3]

    def issue(c, par):
        pltpu.make_async_copy(ent_hbm.at[idx_h_v.at[pl.ds(c * C, C)]], hbuf[par], shs[par]).start()
        pltpu.make_async_copy(ent_hbm.at[idx_t_v.at[pl.ds(c * C, C)]], tbuf[par], sts[par]).start()

    cp_r.wait()
    for par in range(NBUF):
        issue(par, par)
    rv = r_v[pl.ds(0, 16)][0]
    pltpu.sync_copy(rel_hbm.at[pl.ds(rv, 1)], rel_v)

    lane = lax.iota(jnp.int32, 16)
    folds = [((lane & ~(gw - 1)) | ((lane + gw // 2) & (gw - 1)), lane & (gw // 2) == 0)
             for gw in (16, 8, 4, 2)]

    def compute(c, hv_ref, tv_ref):
        @plsc.parallel_loop(0, C // 16)
        def group_body(g):
            row0 = g * 16

            def jbody(j, vecs):
                col = pl.ds(j * 16, 16)
                rj = rel_v[0, col]
                return [vecs[k]
                        + hv_ref[row0 + _BITREV[k], col]
                        * tv_ref[row0 + _BITREV[k], col] * rj
                        for k in range(16)]

            vecs = lax.fori_loop(
                0, LJ, jbody, [jnp.zeros((16,), jnp.float32)] * 16)
            for fidx, mask in folds:
                nxt = []
                for i in range(0, len(vecs), 2):
                    xf = vecs[i] + _permute(vecs[i], fidx)
                    yf = vecs[i + 1] + _permute(vecs[i + 1], fidx)
                    nxt.append(jnp.where(mask, xf, yf))
                vecs = nxt
            out_v[pl.ds(c * C + row0, 16)] = vecs[0]

    def ring_body(qq, carry):
        for par in range(NBUF):
            c = qq * NBUF + par
            pltpu.make_async_copy(ent_hbm.at[idx_h_v.at[pl.ds(c * C, C)]], hbuf[par], shs[par]).wait()
            pltpu.make_async_copy(ent_hbm.at[idx_t_v.at[pl.ds(c * C, C)]], tbuf[par], sts[par]).wait()
            compute(c, hbuf[par], tbuf[par])

            @pl.when(c + NBUF < NCH)
            def _(c=c, par=par):
                issue(c + NBUF, par)
        return carry

    lax.fori_loop(0, NCH // NBUF, ring_body, 0)

    pltpu.sync_copy(out_v, out_hbm.at[pl.ds(base, BPW)])


@jax.jit
def _distmult_sc(ent_embeddings, idx_h, idx_t, rel_embeddings, r_arr):
    mesh = plsc.VectorSubcoreMesh(core_axis_name="c", subcore_axis_name="s")
    fn = pl.kernel(
        _sc_body,
        out_type=jax.ShapeDtypeStruct((B,), jnp.float32),
        mesh=mesh,
        scratch_types=(
            [pltpu.VMEM((BPW,), jnp.int32)] * 2
            + [pltpu.VMEM((C, D), jnp.float32)] * (2 * NBUF)
            + [pltpu.VMEM((1, D), jnp.float32),
               pltpu.VMEM((16,), jnp.int32),
               pltpu.VMEM((BPW,), jnp.float32)]
            + [pltpu.SemaphoreType.DMA] * (2 * NBUF)
        ),
    )
    return fn(ent_embeddings, idx_h, idx_t, rel_embeddings, r_arr)


def kernel(predict_h, predict_t, r, ent_embeddings, rel_embeddings):
    r_arr = jnp.full((16,), r, dtype=jnp.int32)
    return _distmult_sc(ent_embeddings, predict_h, predict_t,
                        rel_embeddings, r_arr)
